# baseline (device time: 85261 ns/iter reference)
import jax
import jax.numpy as jnp
from jax import lax
from jax.experimental import pallas as pl
from jax.experimental.pallas import tpu as pltpu

N_DEV = 4
EPS = 1e-5
BM = 512


def _partial_stats(x):
    m, n = x.shape

    def body(x_ref, s_ref, q_ref):
        xb = x_ref[...]
        s_ref[...] = jnp.sum(xb, axis=1, keepdims=True)
        q_ref[...] = jnp.sum(xb * xb, axis=1, keepdims=True)

    return pl.pallas_call(
        body,
        grid=(m // BM,),
        in_specs=[pl.BlockSpec((BM, n), lambda i: (i, 0))],
        out_specs=[
            pl.BlockSpec((BM, 1), lambda i: (i, 0)),
            pl.BlockSpec((BM, 1), lambda i: (i, 0)),
        ],
        out_shape=[
            jax.ShapeDtypeStruct((m, 1), jnp.float32),
            jax.ShapeDtypeStruct((m, 1), jnp.float32),
        ],
    )(x)


def _allreduce_stats(p, n_global):
    _, m = p.shape

    def body(p_ref, out_ref, comm_ref, send_sems, recv_sems):
        my = lax.axis_index("i")
        left = (my + N_DEV - 1) % N_DEV
        right = (my + 1) % N_DEV

        barrier_sem = pltpu.get_barrier_semaphore()
        for nbr in (left, right):
            pl.semaphore_signal(
                barrier_sem, inc=1,
                device_id=(nbr,), device_id_type=pl.DeviceIdType.MESH,
            )
        pl.semaphore_wait(barrier_sem, 2)

        comm_ref[0, :, :] = p_ref[...]
        for h in range(N_DEV - 1):
            rdma = pltpu.make_async_remote_copy(
                src_ref=comm_ref.at[h],
                dst_ref=comm_ref.at[h + 1],
                send_sem=send_sems.at[h],
                recv_sem=recv_sems.at[h],
                device_id=(right,),
                device_id_type=pl.DeviceIdType.MESH,
            )
            rdma.start()
            rdma.wait()

        total = (
            comm_ref[0, :, :] + comm_ref[1, :, :]
            + comm_ref[2, :, :] + comm_ref[3, :, :]
        )
        mean = total[0:1, :] / n_global
        var = total[1:2, :] / n_global - mean * mean
        rstd = lax.rsqrt(var + EPS)
        out_ref[0:1, :] = mean
        out_ref[1:2, :] = rstd

    return pl.pallas_call(
        body,
        out_shape=jax.ShapeDtypeStruct((2, m), jnp.float32),
        in_specs=[pl.BlockSpec(memory_space=pltpu.VMEM)],
        out_specs=pl.BlockSpec(memory_space=pltpu.VMEM),
        scratch_shapes=[
            pltpu.VMEM((N_DEV, 2, m), jnp.float32),
            pltpu.SemaphoreType.DMA((N_DEV - 1,)),
            pltpu.SemaphoreType.DMA((N_DEV - 1,)),
        ],
        compiler_params=pltpu.CompilerParams(collective_id=0),
    )(p)


def _normalize(x, mean, rstd, gamma, beta):
    m, n = x.shape

    def body(x_ref, mu_ref, r_ref, g_ref, b_ref, o_ref):
        o_ref[...] = (
            (x_ref[...] - mu_ref[...]) * r_ref[...] * g_ref[...] + b_ref[...]
        )

    return pl.pallas_call(
        body,
        grid=(m // BM,),
        in_specs=[
            pl.BlockSpec((BM, n), lambda i: (i, 0)),
            pl.BlockSpec((BM, 1), lambda i: (i, 0)),
            pl.BlockSpec((BM, 1), lambda i: (i, 0)),
            pl.BlockSpec((1, n), lambda i: (0, 0)),
            pl.BlockSpec((1, n), lambda i: (0, 0)),
        ],
        out_specs=pl.BlockSpec((BM, n), lambda i: (i, 0)),
        out_shape=jax.ShapeDtypeStruct((m, n), jnp.float32),
    )(x, mean, rstd, gamma, beta)


def kernel(x, gamma, beta):
    m, n_loc = x.shape
    n_global = n_loc * N_DEV
    psum, psq = _partial_stats(x)
    packed = jnp.concatenate([psum, psq], axis=1).T
    stats = _allreduce_stats(packed, n_global)
    mean = stats[0:1, :].T
    rstd = stats[1:2, :].T
    return _normalize(
        x, mean, rstd, gamma.reshape(1, n_loc), beta.reshape(1, n_loc)
    )
